# baseline (speedup 1.0000x reference)
import functools

import jax
import jax.numpy as jnp
from jax.experimental import pallas as pl
from jax.experimental.pallas import tpu as pltpu

_K = 128
_D = 32
_L = 8
_B = 4096
_NB = 4096
_BB = _B // _L


def _body(x_ref, cb_ref, cbt_ref, pol_ref, qnt_ref, cset_ref, scratch, sem):
    l = pl.program_id(0)
    cb = cb_ref[...]
    cbt = cbt_ref[...]

    @pl.when(l == 0)
    def _fill_and_fire():
        scratch[...] = jnp.broadcast_to(cbt[None], scratch.shape)
        for j in range(_L):
            pltpu.make_async_copy(
                scratch, cset_ref.at[pl.ds(j * _BB, _BB)], sem).start()

    x = x_ref[0]
    prod = jax.lax.dot_general(
        cb, x, (((1,), (0,)), ((), ())),
        preferred_element_type=jnp.float32)
    dist = (jnp.sum(x * x, axis=0, keepdims=True)
            + jnp.sum(cb * cb, axis=1, keepdims=True)) - 2.0 * prod
    mins = jnp.min(dist, axis=0, keepdims=True)
    iota = jax.lax.broadcasted_iota(jnp.int32, dist.shape, 0)
    idx = jnp.min(jnp.where(dist == mins, iota, _K), axis=0, keepdims=True)
    onehot = (iota == idx).astype(jnp.float32)
    q = jax.lax.dot_general(
        cbt, onehot, (((1,), (0,)), ((), ())),
        preferred_element_type=jnp.float32)
    pol_ref[0] = q
    qnt_ref[0] = q

    @pl.when(l == _L - 1)
    def _drain():
        for j in range(_L):
            pltpu.make_async_copy(
                scratch, cset_ref.at[pl.ds(j * _BB, _BB)], sem).wait()


@functools.partial(jax.jit, static_argnames=())
def kernel(latent, codebook):
    lat_t = latent.transpose(1, 2, 0)
    cbt = codebook.T
    pol, qnt, cset_t = pl.pallas_call(
        _body,
        grid=(_L,),
        in_specs=[
            pl.BlockSpec((1, _D, _NB), lambda l: (l, 0, 0)),
            pl.BlockSpec((_K, _D), lambda l: (0, 0)),
            pl.BlockSpec((_D, _K), lambda l: (0, 0)),
        ],
        out_specs=[
            pl.BlockSpec((1, _D, _NB), lambda l: (l, 0, 0)),
            pl.BlockSpec((1, _D, _NB), lambda l: (l, 0, 0)),
            pl.BlockSpec(memory_space=pl.ANY),
        ],
        out_shape=[
            jax.ShapeDtypeStruct((_L, _D, _B), jnp.float32),
            jax.ShapeDtypeStruct((_L, _D, _B), jnp.float32),
            jax.ShapeDtypeStruct((_B, _D, _K), jnp.float32),
        ],
        scratch_shapes=[
            pltpu.VMEM((_BB, _D, _K), jnp.float32),
            pltpu.SemaphoreType.DMA,
        ],
        compiler_params=pltpu.CompilerParams(
            dimension_semantics=("arbitrary",),
        ),
    )(lat_t, codebook, cbt)
    pol = pol.transpose(2, 0, 1)
    qnt = qnt.transpose(2, 0, 1)
    return (pol, qnt, cset_t.transpose(0, 2, 1))
